# baseline (speedup 1.0000x reference)
import jax
import jax.numpy as jnp
from jax import lax
from jax.experimental import pallas as pl
from jax.experimental.pallas import tpu as pltpu

T = 2048
D = 1024
E = 8
FFN = 512
TOPK = 2
NGROUP = 4
EPG = E // NGROUP
SFFN = 512
SCALE = 2.5

TM = 256
NT = T // TM


def _silu(x):
    return x * jax.nn.sigmoid(x)


def _routing(scores, sb, gsum):
    n = scores.shape[0]
    eidx = lax.broadcasted_iota(jnp.int32, (n, E), 1)
    gid = eidx // EPG
    giota = lax.broadcasted_iota(jnp.int32, (n, NGROUP), 1)

    neg = jnp.float32(-jnp.inf)
    big = jnp.int32(NGROUP)

    m1 = jnp.max(gsum, axis=1, keepdims=True)
    g1 = jnp.min(jnp.where(gsum == m1, giota, big), axis=1, keepdims=True)
    gsum2 = jnp.where(giota == g1, neg, gsum)
    m2 = jnp.max(gsum2, axis=1, keepdims=True)
    g2 = jnp.min(jnp.where(gsum2 == m2, giota, big), axis=1, keepdims=True)

    group_ok = (gid == g1) | (gid == g2)
    tmp = jnp.where(group_ok, sb, jnp.float32(0.0))

    ebig = jnp.int32(E)
    t1 = jnp.max(tmp, axis=1, keepdims=True)
    e1 = jnp.min(jnp.where(tmp == t1, eidx, ebig), axis=1, keepdims=True)
    tmp2 = jnp.where(eidx == e1, neg, tmp)
    t2 = jnp.max(tmp2, axis=1, keepdims=True)
    e2 = jnp.min(jnp.where(tmp2 == t2, eidx, ebig), axis=1, keepdims=True)

    sel1 = eidx == e1
    sel2 = eidx == e2
    w1 = jnp.sum(jnp.where(sel1, scores, 0.0), axis=1, keepdims=True)
    w2 = jnp.sum(jnp.where(sel2, scores, 0.0), axis=1, keepdims=True)
    denom = w1 + w2
    combine = (jnp.where(sel1, w1, 0.0) + jnp.where(sel2, w2, 0.0)) / denom
    return combine


def _moe_body(x_ref, scores_ref, sb_ref, gsum_ref, wgu_ref, wd_ref, sgu_ref,
              sd_ref, out_ref, xbf, wgubf, wdbf, sgubf, sdbf, comb):
    e = pl.program_id(0)
    t = pl.program_id(1)

    @pl.when((e == 0) & (t == 0))
    def _prep():
        xbf[...] = x_ref[...].astype(jnp.bfloat16)
        sgubf[...] = sgu_ref[...].astype(jnp.bfloat16)
        sdbf[...] = sd_ref[...].astype(jnp.bfloat16)
        comb[...] = _routing(scores_ref[...], sb_ref[...], gsum_ref[...])

    @pl.when(t == 0)
    def _cast_expert():
        wgubf[...] = wgu_ref[0].astype(jnp.bfloat16)
        wdbf[...] = wd_ref[0].astype(jnp.bfloat16)

    xs = xbf[pl.ds(t * TM, TM), :]

    gue = lax.dot_general(xs, wgubf[...], (((1,), (1,)), ((), ())),
                          preferred_element_type=jnp.float32)
    he = (_silu(gue[:, :FFN]) * gue[:, FFN:]).astype(jnp.bfloat16)
    ye = lax.dot_general(he, wdbf[...], (((1,), (1,)), ((), ())),
                         preferred_element_type=jnp.float32)
    ct = comb[pl.ds(t * TM, TM), :]
    ei = lax.broadcasted_iota(jnp.int32, (TM, E), 1)
    w = jnp.sum(jnp.where(ei == e, ct, 0.0), axis=1, keepdims=True)
    routed = (SCALE * w) * ye

    @pl.when(e == 0)
    def _with_shared():
        gu = lax.dot_general(xs, sgubf[...], (((1,), (1,)), ((), ())),
                             preferred_element_type=jnp.float32)
        h = (_silu(gu[:, :SFFN]) * gu[:, SFFN:]).astype(jnp.bfloat16)
        shared = lax.dot_general(h, sdbf[...], (((1,), (1,)), ((), ())),
                                 preferred_element_type=jnp.float32)
        out_ref[pl.ds(t * TM, TM), :] = shared + routed

    @pl.when(e > 0)
    def _accum():
        out_ref[pl.ds(t * TM, TM), :] += routed


@jax.jit
def _moe(hidden_states, scores, sb, gsum, w_gate_up, w_down, s_gate_up,
         s_down):
    return pl.pallas_call(
        _moe_body,
        grid=(E, NT),
        in_specs=[
            pl.BlockSpec((T, D), lambda e, t: (0, 0)),
            pl.BlockSpec((T, E), lambda e, t: (0, 0)),
            pl.BlockSpec((T, E), lambda e, t: (0, 0)),
            pl.BlockSpec((T, NGROUP), lambda e, t: (0, 0)),
            pl.BlockSpec((1, 2 * FFN, D), lambda e, t: (e, 0, 0)),
            pl.BlockSpec((1, D, FFN), lambda e, t: (e, 0, 0)),
            pl.BlockSpec((2 * SFFN, D), lambda e, t: (0, 0)),
            pl.BlockSpec((D, SFFN), lambda e, t: (0, 0)),
        ],
        out_specs=pl.BlockSpec((T, D), lambda e, t: (0, 0)),
        out_shape=jax.ShapeDtypeStruct((T, D), jnp.float32),
        scratch_shapes=[
            pltpu.VMEM((T, D), jnp.bfloat16),
            pltpu.VMEM((2 * FFN, D), jnp.bfloat16),
            pltpu.VMEM((D, FFN), jnp.bfloat16),
            pltpu.VMEM((2 * SFFN, D), jnp.bfloat16),
            pltpu.VMEM((D, SFFN), jnp.bfloat16),
            pltpu.VMEM((T, E), jnp.float32),
        ],
        compiler_params=pltpu.CompilerParams(
            vmem_limit_bytes=60 * 1024 * 1024),
    )(hidden_states, scores, sb, gsum, w_gate_up, w_down, s_gate_up, s_down)


def kernel(hidden_states, gate_w, corr_bias, w_gate_up, w_down, s_gate_up,
           s_down):
    router_logits = hidden_states.astype(jnp.float32) @ gate_w.T
    scores = jax.nn.sigmoid(router_logits)
    sb = scores + corr_bias[None, :]
    gsum = lax.top_k(sb.reshape(T, NGROUP, EPG), 2)[0].sum(axis=-1)
    return _moe(hidden_states, scores, sb, gsum, w_gate_up, w_down,
                s_gate_up, s_down)


# speedup vs baseline: 1.3147x; 1.3147x over previous
import jax
import jax.numpy as jnp
from jax import lax
from jax.experimental import pallas as pl
from jax.experimental.pallas import tpu as pltpu

T = 2048
D = 1024
E = 8
FFN = 512
TOPK = 2
NGROUP = 4
EPG = E // NGROUP
SFFN = 512
SCALE = 2.5

TM = 256
NT = T // TM


def _silu(x):
    return x * jax.nn.sigmoid(x)


def _routing(scores, sb, gsum):
    n = scores.shape[0]
    eidx = lax.broadcasted_iota(jnp.int32, (n, E), 1)
    gid = eidx // EPG
    giota = lax.broadcasted_iota(jnp.int32, (n, NGROUP), 1)

    neg = jnp.float32(-jnp.inf)
    big = jnp.int32(NGROUP)

    m1 = jnp.max(gsum, axis=1, keepdims=True)
    g1 = jnp.min(jnp.where(gsum == m1, giota, big), axis=1, keepdims=True)
    gsum2 = jnp.where(giota == g1, neg, gsum)
    m2 = jnp.max(gsum2, axis=1, keepdims=True)
    g2 = jnp.min(jnp.where(gsum2 == m2, giota, big), axis=1, keepdims=True)

    group_ok = (gid == g1) | (gid == g2)
    tmp = jnp.where(group_ok, sb, jnp.float32(0.0))

    ebig = jnp.int32(E)
    t1 = jnp.max(tmp, axis=1, keepdims=True)
    e1 = jnp.min(jnp.where(tmp == t1, eidx, ebig), axis=1, keepdims=True)
    tmp2 = jnp.where(eidx == e1, neg, tmp)
    t2 = jnp.max(tmp2, axis=1, keepdims=True)
    e2 = jnp.min(jnp.where(tmp2 == t2, eidx, ebig), axis=1, keepdims=True)

    sel1 = eidx == e1
    sel2 = eidx == e2
    w1 = jnp.sum(jnp.where(sel1, scores, 0.0), axis=1, keepdims=True)
    w2 = jnp.sum(jnp.where(sel2, scores, 0.0), axis=1, keepdims=True)
    denom = w1 + w2
    combine = (jnp.where(sel1, w1, 0.0) + jnp.where(sel2, w2, 0.0)) / denom
    return combine


def _moe_body(x_ref, scores_ref, sb_ref, gsum_ref, wgu_ref, wd_ref, sgu_ref,
              sd_ref, out_ref):
    xs = x_ref[...]

    combine = _routing(scores_ref[...], sb_ref[...], gsum_ref[...])
    cs = SCALE * combine

    gu = lax.dot_general(xs, sgu_ref[...], (((1,), (0,)), ((), ())),
                         preferred_element_type=jnp.float32)
    h = (_silu(gu[:, :SFFN]) * gu[:, SFFN:]).astype(jnp.bfloat16)
    acc = lax.dot_general(h, sd_ref[...], (((1,), (0,)), ((), ())),
                          preferred_element_type=jnp.float32)

    for e in range(E):
        gue = lax.dot_general(xs, wgu_ref[e], (((1,), (0,)), ((), ())),
                              preferred_element_type=jnp.float32)
        he = (_silu(gue[:, :FFN]) * gue[:, FFN:]).astype(jnp.bfloat16)
        ye = lax.dot_general(he, wd_ref[e], (((1,), (0,)), ((), ())),
                             preferred_element_type=jnp.float32)
        acc = acc + cs[:, e:e + 1] * ye

    out_ref[...] = acc


@jax.jit
def _moe(x_bf, scores, sb, gsum, wgu_t, wd_t, sgu_t, sd_t):
    return pl.pallas_call(
        _moe_body,
        grid=(NT,),
        in_specs=[
            pl.BlockSpec((TM, D), lambda t: (t, 0)),
            pl.BlockSpec((TM, E), lambda t: (t, 0)),
            pl.BlockSpec((TM, E), lambda t: (t, 0)),
            pl.BlockSpec((TM, NGROUP), lambda t: (t, 0)),
            pl.BlockSpec((E, D, 2 * FFN), lambda t: (0, 0, 0)),
            pl.BlockSpec((E, FFN, D), lambda t: (0, 0, 0)),
            pl.BlockSpec((D, 2 * SFFN), lambda t: (0, 0)),
            pl.BlockSpec((SFFN, D), lambda t: (0, 0)),
        ],
        out_specs=pl.BlockSpec((TM, D), lambda t: (t, 0)),
        out_shape=jax.ShapeDtypeStruct((T, D), jnp.float32),
        compiler_params=pltpu.CompilerParams(
            vmem_limit_bytes=60 * 1024 * 1024),
    )(x_bf, scores, sb, gsum, wgu_t, wd_t, sgu_t, sd_t)


def kernel(hidden_states, gate_w, corr_bias, w_gate_up, w_down, s_gate_up,
           s_down):
    router_logits = hidden_states.astype(jnp.float32) @ gate_w.T
    scores = jax.nn.sigmoid(router_logits)
    sb = scores + corr_bias[None, :]
    gsum = lax.top_k(sb.reshape(T, NGROUP, EPG), 2)[0].sum(axis=-1)

    bf = jnp.bfloat16
    x_bf = hidden_states.astype(bf)
    wgu_t = w_gate_up.astype(bf).transpose(0, 2, 1)
    wd_t = w_down.astype(bf).transpose(0, 2, 1)
    sgu_t = s_gate_up.astype(bf).T
    sd_t = s_down.astype(bf).T
    return _moe(x_bf, scores, sb, gsum, wgu_t, wd_t, sgu_t, sd_t)


# speedup vs baseline: 6.8688x; 5.2248x over previous
import jax
import jax.numpy as jnp
from jax import lax
from jax.experimental import pallas as pl
from jax.experimental.pallas import tpu as pltpu

T = 2048
D = 1024
E = 8
FFN = 512
TOPK = 2
NGROUP = 4
EPG = E // NGROUP
SFFN = 512
SCALE = 2.5

TM = 256
NT = T // TM


def _silu(x):
    return x * jax.nn.sigmoid(x)


def _routing(scores, sb, gsum):
    n = scores.shape[0]
    eidx = lax.broadcasted_iota(jnp.int32, (n, E), 1)
    gid = eidx // EPG
    giota = lax.broadcasted_iota(jnp.int32, (n, NGROUP), 1)

    neg = jnp.float32(-jnp.inf)
    big = jnp.int32(NGROUP)

    m1 = jnp.max(gsum, axis=1, keepdims=True)
    g1 = jnp.min(jnp.where(gsum == m1, giota, big), axis=1, keepdims=True)
    gsum2 = jnp.where(giota == g1, neg, gsum)
    m2 = jnp.max(gsum2, axis=1, keepdims=True)
    g2 = jnp.min(jnp.where(gsum2 == m2, giota, big), axis=1, keepdims=True)

    group_ok = (gid == g1) | (gid == g2)
    tmp = jnp.where(group_ok, sb, jnp.float32(0.0))

    ebig = jnp.int32(E)
    t1 = jnp.max(tmp, axis=1, keepdims=True)
    e1 = jnp.min(jnp.where(tmp == t1, eidx, ebig), axis=1, keepdims=True)
    tmp2 = jnp.where(eidx == e1, neg, tmp)
    t2 = jnp.max(tmp2, axis=1, keepdims=True)
    e2 = jnp.min(jnp.where(tmp2 == t2, eidx, ebig), axis=1, keepdims=True)

    sel1 = eidx == e1
    sel2 = eidx == e2
    w1 = jnp.sum(jnp.where(sel1, scores, 0.0), axis=1, keepdims=True)
    w2 = jnp.sum(jnp.where(sel2, scores, 0.0), axis=1, keepdims=True)
    denom = w1 + w2
    combine = (jnp.where(sel1, w1, 0.0) + jnp.where(sel2, w2, 0.0)) / denom
    return combine


def _moe_body(x_ref, scores_ref, sb_ref, gsum_ref, wgu_ref, wd_ref, sgu_ref,
              sd_ref, out_ref):
    xs = x_ref[...]

    combine = _routing(scores_ref[...], sb_ref[...], gsum_ref[...])
    cs = SCALE * combine

    gu = lax.dot_general(xs, sgu_ref[...], (((1,), (0,)), ((), ())),
                         preferred_element_type=jnp.float32)
    h = (_silu(gu[:, :SFFN]) * gu[:, SFFN:]).astype(jnp.bfloat16)
    acc = lax.dot_general(h, sd_ref[...], (((1,), (0,)), ((), ())),
                          preferred_element_type=jnp.float32)

    for e in range(E):
        gue = lax.dot_general(xs, wgu_ref[e], (((1,), (0,)), ((), ())),
                              preferred_element_type=jnp.float32)
        he = (_silu(gue[:, :FFN]) * gue[:, FFN:]).astype(jnp.bfloat16)
        ye = lax.dot_general(he, wd_ref[e], (((1,), (0,)), ((), ())),
                             preferred_element_type=jnp.float32)
        acc = acc + cs[:, e:e + 1] * ye

    out_ref[...] = acc


@jax.jit
def _moe(x_bf, scores, sb, gsum, wgu_t, wd_t, sgu_t, sd_t):
    return pl.pallas_call(
        _moe_body,
        grid=(NT,),
        in_specs=[
            pl.BlockSpec((TM, D), lambda t: (t, 0)),
            pl.BlockSpec((TM, E), lambda t: (t, 0)),
            pl.BlockSpec((TM, E), lambda t: (t, 0)),
            pl.BlockSpec((TM, NGROUP), lambda t: (t, 0)),
            pl.BlockSpec((E, D, 2 * FFN), lambda t: (0, 0, 0)),
            pl.BlockSpec((E, FFN, D), lambda t: (0, 0, 0)),
            pl.BlockSpec((D, 2 * SFFN), lambda t: (0, 0)),
            pl.BlockSpec((SFFN, D), lambda t: (0, 0)),
        ],
        out_specs=pl.BlockSpec((TM, D), lambda t: (t, 0)),
        out_shape=jax.ShapeDtypeStruct((T, D), jnp.float32),
        compiler_params=pltpu.CompilerParams(
            vmem_limit_bytes=60 * 1024 * 1024),
    )(x_bf, scores, sb, gsum, wgu_t, wd_t, sgu_t, sd_t)


def kernel(hidden_states, gate_w, corr_bias, w_gate_up, w_down, s_gate_up,
           s_down):
    router_logits = hidden_states.astype(jnp.float32) @ gate_w.T
    scores = jax.nn.sigmoid(router_logits)
    sb = scores + corr_bias[None, :]
    sba, sbb = sb[:, 0::2], sb[:, 1::2]
    gsum = jnp.maximum(sba, sbb) + jnp.minimum(sba, sbb)

    bf = jnp.bfloat16
    x_bf = hidden_states.astype(bf)
    wgu_t = w_gate_up.astype(bf).transpose(0, 2, 1)
    wd_t = w_down.astype(bf).transpose(0, 2, 1)
    sgu_t = s_gate_up.astype(bf).T
    sd_t = s_down.astype(bf).T
    return _moe(x_bf, scores, sb, gsum, wgu_t, wd_t, sgu_t, sd_t)
